# baseline (speedup 1.0000x reference)
import functools

import jax
import jax.numpy as jnp
from jax import lax
from jax.experimental import pallas as pl
from jax.experimental.pallas import tpu as pltpu
from jax.experimental.pallas import tpu_sc as plsc

T, D, E, K, F = 2048, 768, 8, 2, 512
EP = 128
BM = 128
NBMAX = T * K // BM + E
NPAD = NBMAX * BM
WSW = 128
NC, NS = 2, 16
NW = NC * NS
TPW = T // NW


def _router_plan_body(x_ref, gw_ref, pair_ref, blk_ref, cex_ref):
    lane = lax.broadcasted_iota(jnp.int32, (T, EP), 1)
    logits = jnp.dot(x_ref[...], gw_ref[...],
                     preferred_element_type=jnp.float32)
    logits = jnp.where(lane < E, logits, jnp.float32(-1e30))
    m = jnp.max(logits, axis=1, keepdims=True)
    p = jnp.exp(logits - m)
    p = jnp.where(lane < E, p, 0.0)
    p = p / jnp.sum(p, axis=1, keepdims=True)
    m1 = jnp.max(p, axis=1, keepdims=True)
    a1 = jnp.min(jnp.where(p >= m1, lane, EP), axis=1, keepdims=True)
    p2 = jnp.where(lane == a1, jnp.float32(-1.0), p)
    m2 = jnp.max(p2, axis=1, keepdims=True)
    a2 = jnp.min(jnp.where(p2 >= m2, lane, EP), axis=1, keepdims=True)
    wsum = m1 + m2
    w1, w2 = m1 / wsum, m2 / wsum

    oh = (lane == a1).astype(jnp.float32) + (lane == a2).astype(jnp.float32)
    r = lax.broadcasted_iota(jnp.int32, (BM, BM), 0)
    c = lax.broadcasted_iota(jnp.int32, (BM, BM), 1)
    lstrict = (r > c).astype(jnp.float32)
    off = jnp.zeros((1, EP), jnp.float32)
    for b in range(T // BM):
        xb = oh[b * BM:(b + 1) * BM]
        cex_ref[b * BM:(b + 1) * BM] = off + jnp.dot(
            lstrict, xb, preferred_element_type=jnp.float32)
        off = off + jnp.sum(xb, axis=0, keepdims=True)
    counts = off
    nb = jnp.floor((counts + (BM - 1)) * (1.0 / BM))
    nb = jnp.where(lane[0:1] < E, nb, 0.0)
    re = lax.broadcasted_iota(jnp.int32, (EP, EP), 0)
    ce = lax.broadcasted_iota(jnp.int32, (EP, EP), 1)
    ustrict = (re < ce).astype(jnp.float32)
    bstart = jnp.dot(nb, ustrict, preferred_element_type=jnp.float32)
    pstart = bstart * BM

    cex = cex_ref[...]
    d0 = jnp.sum(jnp.where(lane == a1, pstart + cex, 0.0), axis=1,
                 keepdims=True)
    d1 = jnp.sum(jnp.where(lane == a2, pstart + cex, 0.0), axis=1,
                 keepdims=True)
    pair_ref[...] = (jnp.where(lane == 0, d0, 0.0)
                     + jnp.where(lane == 1, d1, 0.0)
                     + jnp.where(lane == 2, w1, 0.0)
                     + jnp.where(lane == 3, w2, 0.0))

    sub = lax.broadcasted_iota(jnp.int32, (EP, EP), 0).astype(jnp.float32)
    lane2 = lax.broadcasted_iota(jnp.int32, (EP, EP), 1)
    cmp = jnp.where((lane2 < E) & (sub >= bstart), 1.0, 0.0)
    be = jnp.sum(cmp, axis=1, keepdims=True) - 1.0
    be = jnp.clip(be, 0.0, float(E - 1))
    tot = jnp.sum(nb, axis=1, keepdims=True)
    valid = (sub[:, 0:1] < tot).astype(jnp.float32)
    lane3 = lax.broadcasted_iota(jnp.int32, (EP, EP), 1)
    blk_ref[...] = (jnp.where(lane3 == 0, be, 0.0)
                    + jnp.where(lane3 == 1, valid, 0.0))


def _router_plan(x, gw_pad):
    return pl.pallas_call(
        _router_plan_body,
        out_shape=[jax.ShapeDtypeStruct((T, EP), jnp.float32),
                   jax.ShapeDtypeStruct((EP, EP), jnp.float32)],
        scratch_shapes=[pltpu.VMEM((T, EP), jnp.float32)],
    )(x, gw_pad)


def _sc_dispatch_body(x_hbm, d0_hbm, d1_hbm, w1_hbm, w2_hbm, xs_hbm, ws_hbm,
                      rows_v, d0_v, d1_v, w1_v, w2_v, s0, s1, s2, s3):
    wid = lax.axis_index("s") * NC + lax.axis_index("c")
    base = wid * TPW
    pltpu.sync_copy(x_hbm.at[pl.ds(base, TPW)], rows_v)
    pltpu.sync_copy(d0_hbm.at[pl.ds(base, TPW)], d0_v)
    pltpu.sync_copy(d1_hbm.at[pl.ds(base, TPW)], d1_v)
    pltpu.sync_copy(w1_hbm.at[pl.ds(base, TPW)], w1_v)
    pltpu.sync_copy(w2_hbm.at[pl.ds(base, TPW)], w2_v)
    c0 = pltpu.async_copy(rows_v, xs_hbm.at[d0_v], s0)
    c1 = pltpu.async_copy(rows_v, xs_hbm.at[d1_v], s1)
    c2 = pltpu.async_copy(w1_v, ws_hbm.at[d0_v], s2)
    c3 = pltpu.async_copy(w2_v, ws_hbm.at[d1_v], s3)
    c0.wait()
    c1.wait()
    c2.wait()
    c3.wait()


def _ffn_body(be_ref, bv_ref, xs_ref, wgu_ref, wd_ref, ws_ref, ys_ref):
    b = pl.program_id(0)

    @pl.when(bv_ref[b] != 0)
    def _():
        x = xs_ref[...].astype(jnp.bfloat16)
        gu = jnp.dot(x, wgu_ref[0].astype(jnp.bfloat16),
                     preferred_element_type=jnp.float32)
        g, u = gu[:, :F], gu[:, F:]
        h = (g * jax.nn.sigmoid(g) * u).astype(jnp.bfloat16)
        y = jnp.dot(h, wd_ref[0].astype(jnp.bfloat16),
                    preferred_element_type=jnp.float32)
        ys_ref[...] = y * ws_ref[:, 0:1]


def _grouped_ffn(be, bv, xs, ws, w_gate_up, w_down):
    grid_spec = pltpu.PrefetchScalarGridSpec(
        num_scalar_prefetch=2,
        grid=(NBMAX,),
        in_specs=[
            pl.BlockSpec((BM, D), lambda b, be, bv: (b, 0)),
            pl.BlockSpec((1, D, 2 * F), lambda b, be, bv: (be[b], 0, 0)),
            pl.BlockSpec((1, F, D), lambda b, be, bv: (be[b], 0, 0)),
            pl.BlockSpec((BM, WSW), lambda b, be, bv: (b, 0)),
        ],
        out_specs=pl.BlockSpec((BM, D), lambda b, be, bv: (b, 0)),
    )
    return pl.pallas_call(
        _ffn_body,
        grid_spec=grid_spec,
        out_shape=jax.ShapeDtypeStruct((NPAD, D), jnp.float32),
    )(be, bv, xs, w_gate_up, w_down, ws)


def _sc_combine_body(ys_hbm, d0_hbm, d1_hbm, out_hbm,
                     a_v, b_v, d0_v, d1_v, s0, s1):
    wid = lax.axis_index("s") * NC + lax.axis_index("c")
    base = wid * TPW
    pltpu.sync_copy(d0_hbm.at[pl.ds(base, TPW)], d0_v)
    pltpu.sync_copy(d1_hbm.at[pl.ds(base, TPW)], d1_v)
    ca = pltpu.async_copy(ys_hbm.at[d0_v], a_v, s0)
    cb = pltpu.async_copy(ys_hbm.at[d1_v], b_v, s1)
    ca.wait()
    cb.wait()

    def row_body(rr, carry):
        for cc in range(D // 16):
            sl = pl.ds(cc * 16, 16)
            a_v[rr, sl] += b_v[rr, sl]
        return carry

    lax.fori_loop(0, TPW, row_body, 0)
    pltpu.sync_copy(a_v, out_hbm.at[pl.ds(base, TPW)])


@functools.lru_cache(maxsize=None)
def _sc_kernels():
    mesh = plsc.VectorSubcoreMesh(core_axis_name="c", subcore_axis_name="s")
    dispatch = pl.kernel(
        _sc_dispatch_body,
        out_type=[jax.ShapeDtypeStruct((NPAD, D), jnp.float32),
                  jax.ShapeDtypeStruct((NPAD, WSW), jnp.float32)],
        mesh=mesh,
        scratch_types=[pltpu.VMEM((TPW, D), jnp.float32),
                       pltpu.VMEM((TPW,), jnp.int32),
                       pltpu.VMEM((TPW,), jnp.int32),
                       pltpu.VMEM((TPW, WSW), jnp.float32),
                       pltpu.VMEM((TPW, WSW), jnp.float32),
                       pltpu.SemaphoreType.DMA,
                       pltpu.SemaphoreType.DMA,
                       pltpu.SemaphoreType.DMA,
                       pltpu.SemaphoreType.DMA],
    )
    combine = pl.kernel(
        _sc_combine_body,
        out_type=jax.ShapeDtypeStruct((T, D), jnp.float32),
        mesh=mesh,
        scratch_types=[pltpu.VMEM((TPW, D), jnp.float32),
                       pltpu.VMEM((TPW, D), jnp.float32),
                       pltpu.VMEM((TPW,), jnp.int32),
                       pltpu.VMEM((TPW,), jnp.int32),
                       pltpu.SemaphoreType.DMA,
                       pltpu.SemaphoreType.DMA],
    )
    return dispatch, combine


@jax.jit
def kernel(hidden_states, gate_w, w_gate_up, w_down):
    _sc_dispatch, _sc_combine = _sc_kernels()
    gw_pad = jnp.pad(gate_w, ((0, 0), (0, EP - E)))
    pair, blk = _router_plan(hidden_states, gw_pad)
    d0 = pair[:, 0].astype(jnp.int32)
    d1 = pair[:, 1].astype(jnp.int32)
    w1 = jnp.broadcast_to(pair[:, 2:3], (T, WSW))
    w2 = jnp.broadcast_to(pair[:, 3:4], (T, WSW))
    be = blk[:NBMAX, 0].astype(jnp.int32)
    bv = blk[:NBMAX, 1].astype(jnp.int32)
    xs, ws = _sc_dispatch(hidden_states, d0, d1, w1, w2)
    ys = _grouped_ffn(be, bv, xs, ws, w_gate_up, w_down)
    return _sc_combine(ys, d0, d1)


# speedup vs baseline: 1.1000x; 1.1000x over previous
import functools

import jax
import jax.numpy as jnp
from jax import lax
from jax.experimental import pallas as pl
from jax.experimental.pallas import tpu as pltpu
from jax.experimental.pallas import tpu_sc as plsc

T, D, E, K, F = 2048, 768, 8, 2, 512
EP = 128
BM = 256
CB = 128
NBMAX = T * K // BM + E
NPAD = NBMAX * BM
WSW = 128
NC, NS = 2, 16
NW = NC * NS
TPW = T // NW


def _router_plan_body(x_ref, gw_ref, pair_ref, blk_ref, cex_ref):
    lane = lax.broadcasted_iota(jnp.int32, (T, EP), 1)
    logits = jnp.dot(x_ref[...], gw_ref[...],
                     preferred_element_type=jnp.float32)
    logits = jnp.where(lane < E, logits, jnp.float32(-1e30))
    m = jnp.max(logits, axis=1, keepdims=True)
    p = jnp.exp(logits - m)
    p = jnp.where(lane < E, p, 0.0)
    p = p / jnp.sum(p, axis=1, keepdims=True)
    m1 = jnp.max(p, axis=1, keepdims=True)
    a1 = jnp.min(jnp.where(p >= m1, lane, EP), axis=1, keepdims=True)
    p2 = jnp.where(lane == a1, jnp.float32(-1.0), p)
    m2 = jnp.max(p2, axis=1, keepdims=True)
    a2 = jnp.min(jnp.where(p2 >= m2, lane, EP), axis=1, keepdims=True)
    wsum = m1 + m2
    w1, w2 = m1 / wsum, m2 / wsum

    oh = (lane == a1).astype(jnp.float32) + (lane == a2).astype(jnp.float32)
    r = lax.broadcasted_iota(jnp.int32, (CB, CB), 0)
    c = lax.broadcasted_iota(jnp.int32, (CB, CB), 1)
    lstrict = (r > c).astype(jnp.float32)
    off = jnp.zeros((1, EP), jnp.float32)
    for b in range(T // CB):
        xb = oh[b * CB:(b + 1) * CB]
        cex_ref[b * CB:(b + 1) * CB] = off + jnp.dot(
            lstrict, xb, preferred_element_type=jnp.float32)
        off = off + jnp.sum(xb, axis=0, keepdims=True)
    counts = off
    nb = jnp.floor((counts + (BM - 1)) * (1.0 / BM))
    nb = jnp.where(lane[0:1] < E, nb, 0.0)
    re = lax.broadcasted_iota(jnp.int32, (EP, EP), 0)
    ce = lax.broadcasted_iota(jnp.int32, (EP, EP), 1)
    ustrict = (re < ce).astype(jnp.float32)
    bstart = jnp.dot(nb, ustrict, preferred_element_type=jnp.float32)
    pstart = bstart * BM

    cex = cex_ref[...]
    d0 = jnp.sum(jnp.where(lane == a1, pstart + cex, 0.0), axis=1,
                 keepdims=True)
    d1 = jnp.sum(jnp.where(lane == a2, pstart + cex, 0.0), axis=1,
                 keepdims=True)
    pair_ref[...] = (jnp.where(lane == 0, d0, 0.0)
                     + jnp.where(lane == 1, d1, 0.0)
                     + jnp.where(lane == 2, w1, 0.0)
                     + jnp.where(lane == 3, w2, 0.0))

    sub = lax.broadcasted_iota(jnp.int32, (EP, EP), 0).astype(jnp.float32)
    lane2 = lax.broadcasted_iota(jnp.int32, (EP, EP), 1)
    cmp = jnp.where((lane2 < E) & (sub >= bstart), 1.0, 0.0)
    be = jnp.sum(cmp, axis=1, keepdims=True) - 1.0
    be = jnp.clip(be, 0.0, float(E - 1))
    tot = jnp.sum(nb, axis=1, keepdims=True)
    valid = (sub[:, 0:1] < tot).astype(jnp.float32)
    lane3 = lax.broadcasted_iota(jnp.int32, (EP, EP), 1)
    blk_ref[...] = (jnp.where(lane3 == 0, be, 0.0)
                    + jnp.where(lane3 == 1, valid, 0.0))


def _router_plan(x, gw_pad):
    return pl.pallas_call(
        _router_plan_body,
        out_shape=[jax.ShapeDtypeStruct((T, EP), jnp.float32),
                   jax.ShapeDtypeStruct((EP, EP), jnp.float32)],
        scratch_shapes=[pltpu.VMEM((T, EP), jnp.float32)],
    )(x, gw_pad)


def _sc_dispatch_body(x_hbm, d0_hbm, d1_hbm, w1_hbm, w2_hbm, xs_hbm, ws_hbm,
                      rows_v, d0_v, d1_v, w1_v, w2_v, s0, s1, s2, s3):
    wid = lax.axis_index("s") * NC + lax.axis_index("c")
    base = wid * TPW
    i0 = pltpu.async_copy(x_hbm.at[pl.ds(base, TPW)], rows_v, s0)
    i1 = pltpu.async_copy(d0_hbm.at[pl.ds(base, TPW)], d0_v, s1)
    i2 = pltpu.async_copy(d1_hbm.at[pl.ds(base, TPW)], d1_v, s2)
    i3 = pltpu.async_copy(w1_hbm.at[pl.ds(base, TPW)], w1_v, s3)
    pltpu.sync_copy(w2_hbm.at[pl.ds(base, TPW)], w2_v)
    i0.wait()
    i1.wait()
    i2.wait()
    i3.wait()
    c0 = pltpu.async_copy(rows_v, xs_hbm.at[d0_v], s0)
    c1 = pltpu.async_copy(rows_v, xs_hbm.at[d1_v], s1)
    c2 = pltpu.async_copy(w1_v, ws_hbm.at[d0_v], s2)
    c3 = pltpu.async_copy(w2_v, ws_hbm.at[d1_v], s3)
    c0.wait()
    c1.wait()
    c2.wait()
    c3.wait()


def _ffn_body(be_ref, bv_ref, xs_ref, wgu_ref, wd_ref, ws_ref, ys_ref):
    b = pl.program_id(0)

    @pl.when(bv_ref[b] != 0)
    def _():
        x = xs_ref[...].astype(jnp.bfloat16)
        gu = jnp.dot(x, wgu_ref[0].astype(jnp.bfloat16),
                     preferred_element_type=jnp.float32)
        g, u = gu[:, :F], gu[:, F:]
        h = (g * jax.nn.sigmoid(g) * u).astype(jnp.bfloat16)
        y = jnp.dot(h, wd_ref[0].astype(jnp.bfloat16),
                    preferred_element_type=jnp.float32)
        ys_ref[...] = y * ws_ref[:, 0:1]


def _grouped_ffn(be, bv, xs, ws, w_gate_up, w_down):
    grid_spec = pltpu.PrefetchScalarGridSpec(
        num_scalar_prefetch=2,
        grid=(NBMAX,),
        in_specs=[
            pl.BlockSpec((BM, D), lambda b, be, bv: (b, 0)),
            pl.BlockSpec((1, D, 2 * F), lambda b, be, bv: (be[b], 0, 0)),
            pl.BlockSpec((1, F, D), lambda b, be, bv: (be[b], 0, 0)),
            pl.BlockSpec((BM, WSW), lambda b, be, bv: (b, 0)),
        ],
        out_specs=pl.BlockSpec((BM, D), lambda b, be, bv: (b, 0)),
    )
    return pl.pallas_call(
        _ffn_body,
        grid_spec=grid_spec,
        out_shape=jax.ShapeDtypeStruct((NPAD, D), jnp.float32),
    )(be, bv, xs, w_gate_up, w_down, ws)


def _sc_combine_body(ys_hbm, d0_hbm, d1_hbm, out_hbm,
                     a_v, b_v, d0_v, d1_v, s0, s1):
    wid = lax.axis_index("s") * NC + lax.axis_index("c")
    base = wid * TPW
    pltpu.sync_copy(d0_hbm.at[pl.ds(base, TPW)], d0_v)
    pltpu.sync_copy(d1_hbm.at[pl.ds(base, TPW)], d1_v)
    ca = pltpu.async_copy(ys_hbm.at[d0_v], a_v, s0)
    cb = pltpu.async_copy(ys_hbm.at[d1_v], b_v, s1)
    ca.wait()
    cb.wait()

    def row_body(rr, carry):
        for cc in range(D // 16):
            sl = pl.ds(cc * 16, 16)
            a_v[rr, sl] += b_v[rr, sl]
        return carry

    lax.fori_loop(0, TPW, row_body, 0)
    pltpu.sync_copy(a_v, out_hbm.at[pl.ds(base, TPW)])


@functools.lru_cache(maxsize=None)
def _sc_kernels():
    mesh = plsc.VectorSubcoreMesh(core_axis_name="c", subcore_axis_name="s")
    dispatch = pl.kernel(
        _sc_dispatch_body,
        out_type=[jax.ShapeDtypeStruct((NPAD, D), jnp.float32),
                  jax.ShapeDtypeStruct((NPAD, WSW), jnp.float32)],
        mesh=mesh,
        scratch_types=[pltpu.VMEM((TPW, D), jnp.float32),
                       pltpu.VMEM((TPW,), jnp.int32),
                       pltpu.VMEM((TPW,), jnp.int32),
                       pltpu.VMEM((TPW, WSW), jnp.float32),
                       pltpu.VMEM((TPW, WSW), jnp.float32),
                       pltpu.SemaphoreType.DMA,
                       pltpu.SemaphoreType.DMA,
                       pltpu.SemaphoreType.DMA,
                       pltpu.SemaphoreType.DMA],
    )
    combine = pl.kernel(
        _sc_combine_body,
        out_type=jax.ShapeDtypeStruct((T, D), jnp.float32),
        mesh=mesh,
        scratch_types=[pltpu.VMEM((TPW, D), jnp.float32),
                       pltpu.VMEM((TPW, D), jnp.float32),
                       pltpu.VMEM((TPW,), jnp.int32),
                       pltpu.VMEM((TPW,), jnp.int32),
                       pltpu.SemaphoreType.DMA,
                       pltpu.SemaphoreType.DMA],
    )
    return dispatch, combine


@jax.jit
def kernel(hidden_states, gate_w, w_gate_up, w_down):
    _sc_dispatch, _sc_combine = _sc_kernels()
    gw_pad = jnp.pad(gate_w, ((0, 0), (0, EP - E)))
    pair, blk = _router_plan(hidden_states, gw_pad)
    d0 = pair[:, 0].astype(jnp.int32)
    d1 = pair[:, 1].astype(jnp.int32)
    w1 = jnp.broadcast_to(pair[:, 2:3], (T, WSW))
    w2 = jnp.broadcast_to(pair[:, 3:4], (T, WSW))
    be = blk[:NBMAX, 0].astype(jnp.int32)
    bv = blk[:NBMAX, 1].astype(jnp.int32)
    xs, ws = _sc_dispatch(hidden_states, d0, d1, w1, w2)
    ys = _grouped_ffn(be, bv, xs, ws, w_gate_up, w_down)
    return _sc_combine(ys, d0, d1)


# speedup vs baseline: 1.2192x; 1.1084x over previous
import functools

import jax
import jax.numpy as jnp
from jax import lax
from jax.experimental import pallas as pl
from jax.experimental.pallas import tpu as pltpu
from jax.experimental.pallas import tpu_sc as plsc

T, D, E, K, F = 2048, 768, 8, 2, 512
EP = 128
BM = 256
CB = 128
NBMAX = T * K // BM + E
NPAD = NBMAX * BM
NC, NS = 2, 16
NW = NC * NS
TPW = T // NW
SL = 16


def _router_plan_body(x_ref, gw_ref, d0f_ref, d1f_ref, wb_ref, blk_ref,
                      cex_ref):
    lane = lax.broadcasted_iota(jnp.int32, (T, EP), 1)
    gw = jnp.pad(gw_ref[...], ((0, 0), (0, EP - E)))
    logits = jnp.dot(x_ref[...], gw, preferred_element_type=jnp.float32)
    logits = jnp.where(lane < E, logits, jnp.float32(-1e30))
    m = jnp.max(logits, axis=1, keepdims=True)
    p = jnp.exp(logits - m)
    p = jnp.where(lane < E, p, 0.0)
    p = p / jnp.sum(p, axis=1, keepdims=True)
    m1 = jnp.max(p, axis=1, keepdims=True)
    a1 = jnp.min(jnp.where(p >= m1, lane, EP), axis=1, keepdims=True)
    p2 = jnp.where(lane == a1, jnp.float32(-1.0), p)
    m2 = jnp.max(p2, axis=1, keepdims=True)
    a2 = jnp.min(jnp.where(p2 >= m2, lane, EP), axis=1, keepdims=True)
    wsum = m1 + m2
    w1, w2 = m1 / wsum, m2 / wsum

    oh = (lane == a1).astype(jnp.float32) + (lane == a2).astype(jnp.float32)
    r = lax.broadcasted_iota(jnp.int32, (CB, CB), 0)
    c = lax.broadcasted_iota(jnp.int32, (CB, CB), 1)
    lstrict = (r > c).astype(jnp.float32)
    off = jnp.zeros((1, EP), jnp.float32)
    for b in range(T // CB):
        xb = oh[b * CB:(b + 1) * CB]
        cex_ref[b * CB:(b + 1) * CB] = off + jnp.dot(
            lstrict, xb, preferred_element_type=jnp.float32)
        off = off + jnp.sum(xb, axis=0, keepdims=True)
    counts = off
    nb = jnp.floor((counts + (BM - 1)) * (1.0 / BM))
    nb = jnp.where(lane[0:1] < E, nb, 0.0)
    re = lax.broadcasted_iota(jnp.int32, (EP, EP), 0)
    ce = lax.broadcasted_iota(jnp.int32, (EP, EP), 1)
    ustrict = (re < ce).astype(jnp.float32)
    bstart = jnp.dot(nb, ustrict, preferred_element_type=jnp.float32)
    pstart = bstart * BM

    cex = cex_ref[...]
    d0 = jnp.sum(jnp.where(lane == a1, pstart + cex, 0.0), axis=1,
                 keepdims=True)
    d1 = jnp.sum(jnp.where(lane == a2, pstart + cex, 0.0), axis=1,
                 keepdims=True)
    d0f_ref[...] = jnp.reshape(d0, (1, T))
    d1f_ref[...] = jnp.reshape(d1, (1, T))
    lane32 = lax.broadcasted_iota(jnp.int32, (T, 2 * SL), 1)
    wb_ref[...] = jnp.where(lane32 < SL, w1, w2)

    sub = lax.broadcasted_iota(jnp.int32, (EP, EP), 0).astype(jnp.float32)
    lane2 = lax.broadcasted_iota(jnp.int32, (EP, EP), 1)
    cmp = jnp.where((lane2 < E) & (sub >= bstart), 1.0, 0.0)
    be = jnp.sum(cmp, axis=1, keepdims=True) - 1.0
    be = jnp.clip(be, 0.0, float(E - 1))
    tot = jnp.sum(nb, axis=1, keepdims=True)
    valid = (sub[:, 0:1] < tot).astype(jnp.float32)
    lane3 = lax.broadcasted_iota(jnp.int32, (EP, EP), 1)
    blk_ref[...] = (jnp.where(lane3 == 0, be, 0.0)
                    + jnp.where(lane3 == 1, valid, 0.0))


def _router_plan(x, gw):
    return pl.pallas_call(
        _router_plan_body,
        out_shape=[jax.ShapeDtypeStruct((1, T), jnp.float32),
                   jax.ShapeDtypeStruct((1, T), jnp.float32),
                   jax.ShapeDtypeStruct((T, 2 * SL), jnp.float32),
                   jax.ShapeDtypeStruct((EP, EP), jnp.float32)],
        scratch_shapes=[pltpu.VMEM((T, EP), jnp.float32)],
    )(x, gw)


def _to_i32(src_v, dst_v):
    for g in range(TPW // SL):
        sl = pl.ds(g * SL, SL)
        dst_v[sl] = src_v[sl].astype(jnp.int32)


def _sc_dispatch_body(x_hbm, d0_hbm, d1_hbm, xs_hbm,
                      rows_v, d0f_v, d1f_v, d0_v, d1_v, s0, s1):
    wid = lax.axis_index("s") * NC + lax.axis_index("c")
    base = wid * TPW
    i0 = pltpu.async_copy(x_hbm.at[pl.ds(base, TPW)], rows_v, s0)
    pltpu.sync_copy(d0_hbm.at[pl.ds(base, TPW)], d0f_v)
    pltpu.sync_copy(d1_hbm.at[pl.ds(base, TPW)], d1f_v)
    _to_i32(d0f_v, d0_v)
    _to_i32(d1f_v, d1_v)
    i0.wait()
    c0 = pltpu.async_copy(rows_v, xs_hbm.at[d0_v], s0)
    c1 = pltpu.async_copy(rows_v, xs_hbm.at[d1_v], s1)
    c0.wait()
    c1.wait()


def _ffn_body(be_ref, bv_ref, xs_ref, wgu_ref, wd_ref, ys_ref):
    b = pl.program_id(0)

    @pl.when(bv_ref[b] != 0)
    def _():
        e = be_ref[b]
        x = xs_ref[...]
        gu = jnp.dot(x, wgu_ref[e], preferred_element_type=jnp.float32)
        g, u = gu[:, :F], gu[:, F:]
        h = g * jax.nn.sigmoid(g) * u
        ys_ref[...] = jnp.dot(h, wd_ref[e], preferred_element_type=jnp.float32)


def _grouped_ffn(be, bv, xs, w_gate_up, w_down):
    grid_spec = pltpu.PrefetchScalarGridSpec(
        num_scalar_prefetch=2,
        grid=(NBMAX,),
        in_specs=[
            pl.BlockSpec((BM, D), lambda b, be, bv: (b, 0)),
            pl.BlockSpec((E, D, 2 * F), lambda b, be, bv: (0, 0, 0)),
            pl.BlockSpec((E, F, D), lambda b, be, bv: (0, 0, 0)),
        ],
        out_specs=pl.BlockSpec((BM, D), lambda b, be, bv: (b, 0)),
    )
    return pl.pallas_call(
        _ffn_body,
        grid_spec=grid_spec,
        out_shape=jax.ShapeDtypeStruct((NPAD, D), jnp.float32),
    )(be, bv, xs, w_gate_up, w_down)


def _sc_combine_body(ys_hbm, d0_hbm, d1_hbm, wb_hbm, out_hbm,
                     a_v, b_v, d0f_v, d1f_v, wb_v, d0_v, d1_v, s0, s1):
    wid = lax.axis_index("s") * NC + lax.axis_index("c")
    base = wid * TPW
    i2 = pltpu.async_copy(wb_hbm.at[pl.ds(base, TPW)], wb_v, s1)
    pltpu.sync_copy(d0_hbm.at[pl.ds(base, TPW)], d0f_v)
    pltpu.sync_copy(d1_hbm.at[pl.ds(base, TPW)], d1f_v)
    _to_i32(d0f_v, d0_v)
    _to_i32(d1f_v, d1_v)
    i2.wait()
    ca = pltpu.async_copy(ys_hbm.at[d0_v], a_v, s0)
    cb = pltpu.async_copy(ys_hbm.at[d1_v], b_v, s1)
    ca.wait()
    cb.wait()

    def row_body(rr, carry):
        w1b = wb_v[rr, pl.ds(0, SL)]
        w2b = wb_v[rr, pl.ds(SL, SL)]
        for cc in range(D // SL):
            sl = pl.ds(cc * SL, SL)
            a_v[rr, sl] = w1b * a_v[rr, sl] + w2b * b_v[rr, sl]
        return carry

    lax.fori_loop(0, TPW, row_body, 0)
    pltpu.sync_copy(a_v, out_hbm.at[pl.ds(base, TPW)])


@functools.lru_cache(maxsize=None)
def _sc_kernels():
    mesh = plsc.VectorSubcoreMesh(core_axis_name="c", subcore_axis_name="s")
    dispatch = pl.kernel(
        _sc_dispatch_body,
        out_type=jax.ShapeDtypeStruct((NPAD, D), jnp.float32),
        mesh=mesh,
        scratch_types=[pltpu.VMEM((TPW, D), jnp.float32),
                       pltpu.VMEM((TPW,), jnp.float32),
                       pltpu.VMEM((TPW,), jnp.float32),
                       pltpu.VMEM((TPW,), jnp.int32),
                       pltpu.VMEM((TPW,), jnp.int32),
                       pltpu.SemaphoreType.DMA,
                       pltpu.SemaphoreType.DMA],
    )
    combine = pl.kernel(
        _sc_combine_body,
        out_type=jax.ShapeDtypeStruct((T, D), jnp.float32),
        mesh=mesh,
        scratch_types=[pltpu.VMEM((TPW, D), jnp.float32),
                       pltpu.VMEM((TPW, D), jnp.float32),
                       pltpu.VMEM((TPW,), jnp.float32),
                       pltpu.VMEM((TPW,), jnp.float32),
                       pltpu.VMEM((TPW, 2 * SL), jnp.float32),
                       pltpu.VMEM((TPW,), jnp.int32),
                       pltpu.VMEM((TPW,), jnp.int32),
                       pltpu.SemaphoreType.DMA,
                       pltpu.SemaphoreType.DMA],
    )
    return dispatch, combine


@jax.jit
def kernel(hidden_states, gate_w, w_gate_up, w_down):
    _sc_dispatch, _sc_combine = _sc_kernels()
    d0f, d1f, wb, blk = _router_plan(hidden_states, gate_w)
    be = blk[:NBMAX, 0].astype(jnp.int32)
    bv = blk[:NBMAX, 1].astype(jnp.int32)
    d0f = d0f.reshape(T)
    d1f = d1f.reshape(T)
    xs = _sc_dispatch(hidden_states, d0f, d1f)
    ys = _grouped_ffn(be, bv, xs, w_gate_up, w_down)
    return _sc_combine(ys, d0f, d1f, wb)
